# Initial kernel scaffold; baseline (speedup 1.0000x reference)
import jax
import jax.numpy as jnp
from jax.experimental import pallas as pl


def kernel(seq, pos_table):
    raise NotImplementedError("write your pallas kernel here")


# speedup vs baseline: 4.0687x; 4.0687x over previous
import jax
import jax.numpy as jnp
from jax.experimental import pallas as pl

_BLOCK_B = 128


def _body(seq_ref, rows_ref, row0_ref, out_ref):
    mask = seq_ref[...] != 0
    out_ref[...] = jnp.where(mask, rows_ref[...], row0_ref[...])


def kernel(seq, pos_table):
    B, L = seq.shape
    D = pos_table.shape[1]
    seq3 = seq.reshape(B, L, 1)
    rows3 = jax.lax.slice(pos_table, (1, 0), (L + 1, D)).reshape(1, L, D)
    row03 = jax.lax.slice(pos_table, (0, 0), (1, D)).reshape(1, 1, D)
    return pl.pallas_call(
        _body,
        grid=(B // _BLOCK_B,),
        in_specs=[
            pl.BlockSpec((_BLOCK_B, L, 1), lambda i: (i, 0, 0)),
            pl.BlockSpec((1, L, D), lambda i: (0, 0, 0)),
            pl.BlockSpec((1, 1, D), lambda i: (0, 0, 0)),
        ],
        out_specs=pl.BlockSpec((_BLOCK_B, L, D), lambda i: (i, 0, 0)),
        out_shape=jax.ShapeDtypeStruct((B, L, D), pos_table.dtype),
    )(seq3, rows3, row03)


# speedup vs baseline: 11.1175x; 2.7324x over previous
import jax
import jax.numpy as jnp
from jax.experimental import pallas as pl

_BLOCK_B = 128


def _body(seq_ref, e_ref, diff_ref, row0_ref, out_ref):
    m = (seq_ref[...] != 0).astype(jnp.bfloat16)
    maskex = jax.lax.dot_general(
        m, e_ref[...], (((1,), (0,)), ((), ())),
        preferred_element_type=jnp.float32)
    out_ref[...] = row0_ref[...] + maskex * diff_ref[...]


def kernel(seq, pos_table):
    B, L = seq.shape
    D = pos_table.shape[1]
    N = L * D
    rows = jax.lax.slice(pos_table, (1, 0), (L + 1, D))
    row0 = jax.lax.slice(pos_table, (0, 0), (1, D))
    eye = jnp.eye(L, dtype=jnp.bfloat16)
    e = jnp.broadcast_to(eye[:, :, None], (L, L, D)).reshape(L, N)
    row0t = jnp.tile(row0, (1, L))
    diff = rows.reshape(1, N) - row0t
    out2d = pl.pallas_call(
        _body,
        grid=(B // _BLOCK_B,),
        in_specs=[
            pl.BlockSpec((_BLOCK_B, L), lambda i: (i, 0)),
            pl.BlockSpec((L, N), lambda i: (0, 0)),
            pl.BlockSpec((1, N), lambda i: (0, 0)),
            pl.BlockSpec((1, N), lambda i: (0, 0)),
        ],
        out_specs=pl.BlockSpec((_BLOCK_B, N), lambda i: (i, 0)),
        out_shape=jax.ShapeDtypeStruct((B, N), pos_table.dtype),
    )(seq, e, diff, row0t)
    return out2d.reshape(B, L, D)
